# baseline (speedup 1.0000x reference)
import functools

import jax
import jax.numpy as jnp
from jax import lax
from jax.experimental import pallas as pl
from jax.experimental.pallas import tpu as pltpu
from jax.experimental.pallas import tpu_sc as plsc

_B, _L, _D = 16, 4096, 512
_D0, _D1, _D2 = 256, 153, 103
_P1, _P2 = 256, 128
_V1, _V2 = 8192, 100000
_NBUF = 4
_CH = 32


def _sc_gather(E0, E1v, E2v, gh0, gh1, gh2):
    info = plsc.get_sparse_core_info()
    nc, ns = info.num_cores, info.num_subcores
    nw = nc * ns
    bpw = _L // nw
    mesh = plsc.VectorSubcoreMesh(core_axis_name="c", subcore_axis_name="s")

    @functools.partial(
        pl.kernel,
        mesh=mesh,
        compiler_params=pltpu.CompilerParams(needs_layout_passes=False),
        out_type=(
            jax.ShapeDtypeStruct((_L, _D0), jnp.float32),
            jax.ShapeDtypeStruct((_L, _P1), jnp.float32),
            jax.ShapeDtypeStruct((_L, _P2), jnp.float32),
        ),
        scratch_types=[
            pltpu.VMEM((bpw,), jnp.int32),
            pltpu.VMEM((bpw,), jnp.int32),
            pltpu.VMEM((bpw,), jnp.int32),
            pltpu.VMEM((bpw, _D0), jnp.float32),
            pltpu.VMEM((_NBUF, 8, _D1), jnp.float32),
            pltpu.VMEM((_NBUF, 8, _D2), jnp.float32),
            pltpu.VMEM((bpw, _P1), jnp.float32),
            pltpu.VMEM((bpw, _P2), jnp.float32),
            pltpu.SemaphoreType.DMA,
        ]
        + [pltpu.SemaphoreType.DMA] * _NBUF
        + [pltpu.SemaphoreType.DMA] * _NBUF
        + [pltpu.SemaphoreType.DMA, pltpu.SemaphoreType.DMA],
    )
    def k(e0_h, e1_h, e2_h, g0_h, g1_h, g2_h, o0_h, o1_h, o2_h,
          i0, i1, i2, r0, b1, b2, x1, x2, sem0, *sems):
        sems1 = sems[:_NBUF]
        sems2 = sems[_NBUF:2 * _NBUF]
        semo1, semo2 = sems[2 * _NBUF], sems[2 * _NBUF + 1]
        wid = lax.axis_index("s") * nc + lax.axis_index("c")
        base = wid * bpw
        pltpu.sync_copy(g0_h.at[pl.ds(base, bpw)], i0)
        pltpu.sync_copy(g1_h.at[pl.ds(base, bpw)], i1)
        pltpu.sync_copy(g2_h.at[pl.ds(base, bpw)], i2)
        c0 = pltpu.async_copy(e0_h.at[i0], r0, sem0)
        lanes = lax.iota(jnp.int32, 16)
        last = bpw - 1

        def gh_at(ibuf, p):
            v = ibuf[pl.ds((p >> 4) * 16, 16)]
            sel = jnp.where(lanes == (p & 15), v, 0)
            return lax.reduce_max(sel, (0,))

        def fire(b, p):
            t1 = gh_at(i1, p) >> 3
            t2 = gh_at(i2, p) >> 3
            cA = pltpu.async_copy(e1_h.at[t1], b1.at[b], sems1[b])
            cB = pltpu.async_copy(e2_h.at[t2], b2.at[b], sems2[b])
            return cA, cB

        def extract(b, p, ibuf, blocks, xbuf, width):
            m = gh_at(ibuf, p) & 7
            msplat = lanes * 0 + m
            bsplat = lanes * 0 + b
            psplat = lanes * 0 + p
            nch = (width + 15) // 16
            for jc in range(nch):
                jvec = lanes + jc * 16
                msk = jvec < width
                val = plsc.load_gather(blocks, [bsplat, msplat, jvec], mask=msk)
                plsc.store_scatter(xbuf, [psplat, jvec], val, mask=msk)

        for b in range(_NBUF):
            fire(b, jnp.int32(b))

        out_cps = []
        for c in range(bpw // _CH):

            def group(g, _, c=c):
                for b in range(_NBUF):
                    p = c * _CH + g * _NBUF + b
                    pltpu.make_async_copy(e1_h.at[0], b1.at[b], sems1[b]).wait()
                    extract(b, p, i1, b1, x1, _D1)
                    pltpu.make_async_copy(e2_h.at[0], b2.at[b], sems2[b]).wait()
                    extract(b, p, i2, b2, x2, _D2)
                    fire(b, jnp.minimum(p + _NBUF, last))
                return _

            lax.fori_loop(0, _CH // _NBUF, group, None)
            out_cps.append(pltpu.async_copy(
                x1.at[pl.ds(c * _CH, _CH)],
                o1_h.at[pl.ds(base + c * _CH, _CH)], semo1))
            out_cps.append(pltpu.async_copy(
                x2.at[pl.ds(c * _CH, _CH)],
                o2_h.at[pl.ds(base + c * _CH, _CH)], semo2))

        for b in range(_NBUF):
            pltpu.make_async_copy(e1_h.at[0], b1.at[b], sems1[b]).wait()
            pltpu.make_async_copy(e2_h.at[0], b2.at[b], sems2[b]).wait()
        for cp in out_cps:
            cp.wait()
        c0.wait()
        pltpu.sync_copy(r0, o0_h.at[pl.ds(base, bpw)])

    return k(E0, E1v, E2v, gh0, gh1, gh2)


_TL = 1024


def _add_body(x_ref, e0_ref, e1_ref, e2_ref, o_ref):
    emb = jnp.concatenate(
        [e0_ref[...], e1_ref[:, : _D1], e2_ref[:, : _D2]], axis=-1
    )
    o_ref[...] = x_ref[...] + emb[None]


def _tc_add(x, e0, e1, e2):
    return pl.pallas_call(
        _add_body,
        grid=(_L // _TL, _B),
        in_specs=[
            pl.BlockSpec((1, _TL, _D), lambda l, b: (b, l, 0)),
            pl.BlockSpec((_TL, _D0), lambda l, b: (l, 0)),
            pl.BlockSpec((_TL, _P1), lambda l, b: (l, 0)),
            pl.BlockSpec((_TL, _P2), lambda l, b: (l, 0)),
        ],
        out_specs=pl.BlockSpec((1, _TL, _D), lambda l, b: (b, l, 0)),
        out_shape=jax.ShapeDtypeStruct((_B, _L, _D), jnp.float32),
        compiler_params=pltpu.CompilerParams(
            dimension_semantics=("arbitrary", "arbitrary")
        ),
    )(x, e0, e1, e2)


def kernel(x, E0, E1, E2, gh0, gh1, gh2):
    E1v = E1.reshape(_V1 // 8, 8, _D1)
    E2v = E2.reshape(_V2 // 8, 8, _D2)
    e0, e1, e2 = _sc_gather(E0, E1v, E2v, gh0, gh1, gh2)
    return _tc_add(x, e0, e1, e2)


# speedup vs baseline: 1.6245x; 1.6245x over previous
import functools

import jax
import jax.numpy as jnp
from jax import lax
from jax.experimental import pallas as pl
from jax.experimental.pallas import tpu as pltpu
from jax.experimental.pallas import tpu_sc as plsc

_B, _L, _D = 16, 4096, 512
_D0, _D1, _D2 = 256, 153, 103
_P1, _P2 = 256, 128
_V1, _V2 = 8192, 100000
_NBUF = 4
_CH = 32


def _sc_gather(E0, E1v, E2v, gh0, gh1, gh2):
    info = plsc.get_sparse_core_info()
    nc, ns = info.num_cores, info.num_subcores
    nw = nc * ns
    bpw = _L // nw
    mesh = plsc.VectorSubcoreMesh(core_axis_name="c", subcore_axis_name="s")

    @functools.partial(
        pl.kernel,
        mesh=mesh,
        compiler_params=pltpu.CompilerParams(needs_layout_passes=False),
        out_type=(
            jax.ShapeDtypeStruct((_L, _D0), jnp.float32),
            jax.ShapeDtypeStruct((_L, _P1), jnp.float32),
            jax.ShapeDtypeStruct((_L, _P2), jnp.float32),
        ),
        scratch_types=[
            pltpu.VMEM((bpw,), jnp.int32),
            pltpu.VMEM((bpw,), jnp.int32),
            pltpu.VMEM((bpw,), jnp.int32),
            pltpu.VMEM((bpw, _D0), jnp.float32),
            pltpu.VMEM((_NBUF, 8, _D1), jnp.float32),
            pltpu.VMEM((_NBUF, 8, _D2), jnp.float32),
            pltpu.VMEM((bpw, _P1), jnp.float32),
            pltpu.VMEM((bpw, _P2), jnp.float32),
            pltpu.SemaphoreType.DMA,
        ]
        + [pltpu.SemaphoreType.DMA] * _NBUF
        + [pltpu.SemaphoreType.DMA] * _NBUF
        + [pltpu.SemaphoreType.DMA, pltpu.SemaphoreType.DMA],
    )
    def k(e0_h, e1_h, e2_h, g0_h, g1_h, g2_h, o0_h, o1_h, o2_h,
          i0, i1, i2, r0, b1, b2, x1, x2, sem0, *sems):
        sems1 = sems[:_NBUF]
        sems2 = sems[_NBUF:2 * _NBUF]
        semo1, semo2 = sems[2 * _NBUF], sems[2 * _NBUF + 1]
        wid = lax.axis_index("s") * nc + lax.axis_index("c")
        base = wid * bpw
        pltpu.sync_copy(g0_h.at[pl.ds(base, bpw)], i0)
        pltpu.sync_copy(g1_h.at[pl.ds(base, bpw)], i1)
        pltpu.sync_copy(g2_h.at[pl.ds(base, bpw)], i2)
        c0 = pltpu.async_copy(e0_h.at[i0], r0, sem0)
        lanes = lax.iota(jnp.int32, 16)
        last = bpw - 1

        def gh_at(ibuf, p):
            v = ibuf[pl.ds((p >> 4) * 16, 16)]
            sel = jnp.where(lanes == (p & 15), v, 0)
            return lax.reduce_max(sel, (0,))

        def fire(b, p):
            t1 = (gh_at(i1, p) >> 3) * 8
            t2 = (gh_at(i2, p) >> 3) * 8
            cA = pltpu.async_copy(e1_h.at[pl.ds(t1, 8)], b1.at[b], sems1[b])
            cB = pltpu.async_copy(e2_h.at[pl.ds(t2, 8)], b2.at[b], sems2[b])
            return cA, cB

        def extract(b, p, ibuf, blocks, xbuf, width):
            m = gh_at(ibuf, p) & 7
            msplat = lanes * 0 + m
            bsplat = lanes * 0 + b
            psplat = lanes * 0 + p
            nch = (width + 15) // 16
            for jc in range(nch):
                jvec = lanes + jc * 16
                msk = jvec < width
                val = plsc.load_gather(blocks, [bsplat, msplat, jvec], mask=msk)
                plsc.store_scatter(xbuf, [psplat, jvec], val, mask=msk)

        for b in range(_NBUF):
            fire(b, jnp.int32(b))

        out_cps = []
        for c in range(bpw // _CH):

            def group(g, _, c=c):
                for b in range(_NBUF):
                    p = c * _CH + g * _NBUF + b
                    pltpu.make_async_copy(e1_h.at[pl.ds(0, 8)], b1.at[b], sems1[b]).wait()
                    extract(b, p, i1, b1, x1, _D1)
                    pltpu.make_async_copy(e2_h.at[pl.ds(0, 8)], b2.at[b], sems2[b]).wait()
                    extract(b, p, i2, b2, x2, _D2)
                    fire(b, jnp.minimum(p + _NBUF, last))
                return _

            lax.fori_loop(0, _CH // _NBUF, group, None)
            out_cps.append(pltpu.async_copy(
                x1.at[pl.ds(c * _CH, _CH)],
                o1_h.at[pl.ds(base + c * _CH, _CH)], semo1))
            out_cps.append(pltpu.async_copy(
                x2.at[pl.ds(c * _CH, _CH)],
                o2_h.at[pl.ds(base + c * _CH, _CH)], semo2))

        for b in range(_NBUF):
            pltpu.make_async_copy(e1_h.at[pl.ds(0, 8)], b1.at[b], sems1[b]).wait()
            pltpu.make_async_copy(e2_h.at[pl.ds(0, 8)], b2.at[b], sems2[b]).wait()
        for cp in out_cps:
            cp.wait()
        c0.wait()
        pltpu.sync_copy(r0, o0_h.at[pl.ds(base, bpw)])

    return k(E0, E1v, E2v, gh0, gh1, gh2)


_TL = 1024


def _add_body(x_ref, e0_ref, e1_ref, e2_ref, o_ref):
    emb = jnp.concatenate(
        [e0_ref[...], e1_ref[:, : _D1], e2_ref[:, : _D2]], axis=-1
    )
    o_ref[...] = x_ref[...] + emb[None]


def _tc_add(x, e0, e1, e2):
    return pl.pallas_call(
        _add_body,
        grid=(_L // _TL, _B),
        in_specs=[
            pl.BlockSpec((1, _TL, _D), lambda l, b: (b, l, 0)),
            pl.BlockSpec((_TL, _D0), lambda l, b: (l, 0)),
            pl.BlockSpec((_TL, _P1), lambda l, b: (l, 0)),
            pl.BlockSpec((_TL, _P2), lambda l, b: (l, 0)),
        ],
        out_specs=pl.BlockSpec((1, _TL, _D), lambda l, b: (b, l, 0)),
        out_shape=jax.ShapeDtypeStruct((_B, _L, _D), jnp.float32),
        compiler_params=pltpu.CompilerParams(
            dimension_semantics=("arbitrary", "arbitrary")
        ),
    )(x, e0, e1, e2)


def kernel(x, E0, E1, E2, gh0, gh1, gh2):
    e0, e1, e2 = _sc_gather(E0, E1, E2, gh0, gh1, gh2)
    return _tc_add(x, e0, e1, e2)


# speedup vs baseline: 1.6699x; 1.0280x over previous
import functools

import jax
import jax.numpy as jnp
from jax import lax
from jax.experimental import pallas as pl
from jax.experimental.pallas import tpu as pltpu
from jax.experimental.pallas import tpu_sc as plsc

_B, _L, _D = 16, 4096, 512
_D0, _D1, _D2 = 256, 153, 103
_P1, _P2 = 256, 128
_V1, _V2 = 8192, 100000
_NBUF = 8
_CH = 32


def _sc_gather(E0, E1v, E2v, gh0, gh1, gh2):
    info = plsc.get_sparse_core_info()
    nc, ns = info.num_cores, info.num_subcores
    nw = nc * ns
    bpw = _L // nw
    mesh = plsc.VectorSubcoreMesh(core_axis_name="c", subcore_axis_name="s")

    @functools.partial(
        pl.kernel,
        mesh=mesh,
        compiler_params=pltpu.CompilerParams(needs_layout_passes=False),
        out_type=(
            jax.ShapeDtypeStruct((_L, _D0), jnp.float32),
            jax.ShapeDtypeStruct((_L, _P1), jnp.float32),
            jax.ShapeDtypeStruct((_L, _P2), jnp.float32),
        ),
        scratch_types=[
            pltpu.VMEM((bpw,), jnp.int32),
            pltpu.VMEM((bpw,), jnp.int32),
            pltpu.VMEM((bpw,), jnp.int32),
            pltpu.VMEM((bpw, _D0), jnp.float32),
            pltpu.VMEM((_NBUF, 8, _D1), jnp.float32),
            pltpu.VMEM((_NBUF, 8, _D2), jnp.float32),
            pltpu.VMEM((bpw, _P1), jnp.float32),
            pltpu.VMEM((bpw, _P2), jnp.float32),
            pltpu.SemaphoreType.DMA,
        ]
        + [pltpu.SemaphoreType.DMA] * _NBUF
        + [pltpu.SemaphoreType.DMA] * _NBUF
        + [pltpu.SemaphoreType.DMA, pltpu.SemaphoreType.DMA],
    )
    def k(e0_h, e1_h, e2_h, g0_h, g1_h, g2_h, o0_h, o1_h, o2_h,
          i0, i1, i2, r0, b1, b2, x1, x2, sem0, *sems):
        sems1 = sems[:_NBUF]
        sems2 = sems[_NBUF:2 * _NBUF]
        semo1, semo2 = sems[2 * _NBUF], sems[2 * _NBUF + 1]
        wid = lax.axis_index("s") * nc + lax.axis_index("c")
        base = wid * bpw
        pltpu.sync_copy(g0_h.at[pl.ds(base, bpw)], i0)
        pltpu.sync_copy(g1_h.at[pl.ds(base, bpw)], i1)
        pltpu.sync_copy(g2_h.at[pl.ds(base, bpw)], i2)
        c0 = pltpu.async_copy(e0_h.at[i0], r0, sem0)
        lanes = lax.iota(jnp.int32, 16)
        last = bpw - 1

        def gh_at(ibuf, p):
            v = ibuf[pl.ds((p >> 4) * 16, 16)]
            sel = jnp.where(lanes == (p & 15), v, 0)
            return lax.reduce_max(sel, (0,))

        def fire(b, p):
            t1 = (gh_at(i1, p) >> 3) * 8
            t2 = (gh_at(i2, p) >> 3) * 8
            cA = pltpu.async_copy(e1_h.at[pl.ds(t1, 8)], b1.at[b], sems1[b])
            cB = pltpu.async_copy(e2_h.at[pl.ds(t2, 8)], b2.at[b], sems2[b])
            return cA, cB

        def extract(b, p, ibuf, blocks, xbuf, width):
            m = gh_at(ibuf, p) & 7
            msplat = lanes * 0 + m
            bsplat = lanes * 0 + b
            psplat = lanes * 0 + p
            nch = (width + 15) // 16
            for jc in range(nch):
                jvec = lanes + jc * 16
                msk = jvec < width
                val = plsc.load_gather(blocks, [bsplat, msplat, jvec], mask=msk)
                plsc.store_scatter(xbuf, [psplat, jvec], val, mask=msk)

        for b in range(_NBUF):
            fire(b, jnp.int32(b))

        out_cps = []
        for c in range(bpw // _CH):

            def group(g, _, c=c):
                for b in range(_NBUF):
                    p = c * _CH + g * _NBUF + b
                    pltpu.make_async_copy(e1_h.at[pl.ds(0, 8)], b1.at[b], sems1[b]).wait()
                    extract(b, p, i1, b1, x1, _D1)
                    pltpu.make_async_copy(e2_h.at[pl.ds(0, 8)], b2.at[b], sems2[b]).wait()
                    extract(b, p, i2, b2, x2, _D2)
                    fire(b, jnp.minimum(p + _NBUF, last))
                return _

            lax.fori_loop(0, _CH // _NBUF, group, None)
            out_cps.append(pltpu.async_copy(
                x1.at[pl.ds(c * _CH, _CH)],
                o1_h.at[pl.ds(base + c * _CH, _CH)], semo1))
            out_cps.append(pltpu.async_copy(
                x2.at[pl.ds(c * _CH, _CH)],
                o2_h.at[pl.ds(base + c * _CH, _CH)], semo2))

        for b in range(_NBUF):
            pltpu.make_async_copy(e1_h.at[pl.ds(0, 8)], b1.at[b], sems1[b]).wait()
            pltpu.make_async_copy(e2_h.at[pl.ds(0, 8)], b2.at[b], sems2[b]).wait()
        for cp in out_cps:
            cp.wait()
        c0.wait()
        pltpu.sync_copy(r0, o0_h.at[pl.ds(base, bpw)])

    return k(E0, E1v, E2v, gh0, gh1, gh2)


_TL = 2048


def _add_body(x_ref, e0_ref, e1_ref, e2_ref, o_ref):
    emb = jnp.concatenate(
        [e0_ref[...], e1_ref[:, : _D1], e2_ref[:, : _D2]], axis=-1
    )
    o_ref[...] = x_ref[...] + emb[None]


def _tc_add(x, e0, e1, e2):
    return pl.pallas_call(
        _add_body,
        grid=(_L // _TL, _B),
        in_specs=[
            pl.BlockSpec((1, _TL, _D), lambda l, b: (b, l, 0)),
            pl.BlockSpec((_TL, _D0), lambda l, b: (l, 0)),
            pl.BlockSpec((_TL, _P1), lambda l, b: (l, 0)),
            pl.BlockSpec((_TL, _P2), lambda l, b: (l, 0)),
        ],
        out_specs=pl.BlockSpec((1, _TL, _D), lambda l, b: (b, l, 0)),
        out_shape=jax.ShapeDtypeStruct((_B, _L, _D), jnp.float32),
        compiler_params=pltpu.CompilerParams(
            dimension_semantics=("arbitrary", "arbitrary")
        ),
    )(x, e0, e1, e2)


def kernel(x, E0, E1, E2, gh0, gh1, gh2):
    e0, e1, e2 = _sc_gather(E0, E1, E2, gh0, gh1, gh2)
    return _tc_add(x, e0, e1, e2)
